# baseline (speedup 1.0000x reference)
import jax
import jax.numpy as jnp
from jax.experimental import pallas as pl
from jax.experimental.pallas import tpu as pltpu

_N, _D = 65536, 256
_BM = 2048
_BLK = _N // _BM


def _loss_kernel(za_ref, zb_ref, out_ref):
    b = pl.program_id(0)
    d = jnp.sum(za_ref[...] * zb_ref[...], axis=1, keepdims=True)
    t = jnp.float32(_N - 3) * d + jnp.exp(d)
    partial = jnp.sum(t, axis=0, keepdims=True)

    @pl.when(b == 0)
    def _():
        out_ref[...] = jnp.zeros_like(out_ref)

    out_ref[...] += partial.reshape(1, 1, 1)

    @pl.when(b == _BLK - 1)
    def _():
        out_ref[...] += d[_BM - 1:_BM, :].reshape(1, 1, 1)


def kernel(z_a, z_b):
    out = pl.pallas_call(
        _loss_kernel,
        grid=(_BLK,),
        in_specs=[
            pl.BlockSpec((_BM, _D), lambda b: (b, 0)),
            pl.BlockSpec((_BM, _D), lambda b: (b, 0)),
        ],
        out_specs=pl.BlockSpec((1, 1, 1), lambda b: (0, 0, 0)),
        out_shape=jax.ShapeDtypeStruct((1, 1, 1), jnp.float32),
        compiler_params=pltpu.CompilerParams(
            dimension_semantics=("arbitrary",),
            vmem_limit_bytes=50 * 1024 * 1024,
        ),
        name="contrastive_loss",
    )(z_a, z_b)
    return out[0, 0, 0]


# speedup vs baseline: 1.1382x; 1.1382x over previous
import jax
import jax.numpy as jnp
from jax.experimental import pallas as pl
from jax.experimental.pallas import tpu as pltpu

_N, _D = 65536, 256
_BM = 4096
_BLK = _N // _BM


def _loss_kernel(za_ref, zb_ref, out_ref):
    b = pl.program_id(0)
    d = jnp.sum(za_ref[...] * zb_ref[...], axis=1, keepdims=True)
    t = jnp.float32(_N - 3) * d + jnp.exp(d)
    partial = jnp.sum(t, axis=0, keepdims=True)

    @pl.when(b == 0)
    def _():
        out_ref[...] = jnp.zeros_like(out_ref)

    out_ref[...] += partial.reshape(1, 1, 1)

    @pl.when(b == _BLK - 1)
    def _():
        out_ref[...] += d[_BM - 1:_BM, :].reshape(1, 1, 1)


def kernel(z_a, z_b):
    out = pl.pallas_call(
        _loss_kernel,
        grid=(_BLK,),
        in_specs=[
            pl.BlockSpec((_BM, _D), lambda b: (b, 0)),
            pl.BlockSpec((_BM, _D), lambda b: (b, 0)),
        ],
        out_specs=pl.BlockSpec((1, 1, 1), lambda b: (0, 0, 0)),
        out_shape=jax.ShapeDtypeStruct((1, 1, 1), jnp.float32),
        compiler_params=pltpu.CompilerParams(
            dimension_semantics=("arbitrary",),
            vmem_limit_bytes=50 * 1024 * 1024,
        ),
        name="contrastive_loss",
    )(z_a, z_b)
    return out[0, 0, 0]
